# Initial kernel scaffold; baseline (speedup 1.0000x reference)
import jax
import jax.numpy as jnp
from jax.experimental import pallas as pl


def kernel(h, edge_index, Wsrc, Wdst, attn_l, attn_r, gat_bias, bias, ln_g, ln_b):
    raise NotImplementedError("write your pallas kernel here")


# speedup vs baseline: 20.5260x; 20.5260x over previous
import functools

import jax
import jax.numpy as jnp
from jax import lax
from jax.experimental import pallas as pl
from jax.experimental.pallas import tpu as pltpu
from jax.experimental.pallas import tpu_sc as plsc

N = 10000
NP = 10240
D = 128
E = 320000
NW = 32
EW = E // NW
K = 80
NCH = EW // K
RPT = NP // 16

_f32 = jnp.float32
_i32 = jnp.int32


def _mm_body(l_is_dummy, x_ref, ws_ref, wd_ref, al_ref, ar_ref,
             fs0_ref, fs1_ref, el0_ref, el1_ref, er0_ref, er1_ref):
    xb = x_ref[...]
    fs_refs = (fs0_ref, fs1_ref)
    el_refs = (el0_ref, el1_ref)
    er_refs = (er0_ref, er1_ref)
    for r in range(2):
        fsr = jnp.dot(xb, ws_ref[r], preferred_element_type=_f32)
        fs_refs[r][...] = fsr
        el_refs[r][...] = jnp.sum(fsr * al_ref[r][None, :], axis=1)
        wdr = jnp.sum(wd_ref[r] * ar_ref[r][None, :], axis=1)
        er_refs[r][...] = jnp.sum(xb * wdr[None, :], axis=1)


def _tc_matmul(x, ws, wd, al, ar):
    B = 1024
    grid = NP // B
    out = pl.pallas_call(
        functools.partial(_mm_body, None),
        grid=(grid,),
        in_specs=[
            pl.BlockSpec((B, D), lambda i: (i, 0)),
            pl.BlockSpec((2, D, D), lambda i: (0, 0, 0)),
            pl.BlockSpec((2, D, D), lambda i: (0, 0, 0)),
            pl.BlockSpec((2, D), lambda i: (0, 0)),
            pl.BlockSpec((2, D), lambda i: (0, 0)),
        ],
        out_specs=[
            pl.BlockSpec((B, D), lambda i: (i, 0)),
            pl.BlockSpec((B, D), lambda i: (i, 0)),
            pl.BlockSpec((B,), lambda i: (i,)),
            pl.BlockSpec((B,), lambda i: (i,)),
            pl.BlockSpec((B,), lambda i: (i,)),
            pl.BlockSpec((B,), lambda i: (i,)),
        ],
        out_shape=[
            jax.ShapeDtypeStruct((NP, D), _f32),
            jax.ShapeDtypeStruct((NP, D), _f32),
            jax.ShapeDtypeStruct((NP,), _f32),
            jax.ShapeDtypeStruct((NP,), _f32),
            jax.ShapeDtypeStruct((NP,), _f32),
            jax.ShapeDtypeStruct((NP,), _f32),
        ],
    )(x, ws, wd, al, ar)
    return out


def _sc_edge_body(fs0, fs1, el0, er0, el1, er1, src0, dst0, src1, dst1,
                  u0, s0, u1, s1,
                  el_l, er_l, srcb, dstb, exb, sbuf, rowb,
                  u_sh, s_sh, sem):
    c = lax.axis_index("c")
    s = lax.axis_index("s")
    wid = s * 2 + c

    zv = jnp.zeros((16,), _f32)
    iota = lax.iota(_i32, 16)
    zidx = jnp.zeros((16,), _i32)

    fs_in = (fs0, fs1)
    eler = ((el0, er0), (el1, er1))
    edges = ((src0, dst0), (src1, dst1))
    outs = ((u0, s0), (u1, s1))

    r0 = RPT * s

    for r in range(2):
        plsc.subcore_barrier()

        def _zb(i, _):
            for j in range(8):
                rowb[i, pl.ds(j * 16, 16)] = zv
            sbuf[i, :] = zv
            return _
        lax.fori_loop(0, K, _zb, 0)
        for j in range(RPT // K):
            pltpu.sync_copy(rowb, u_sh.at[pl.ds(r0 + j * K, K)])
            pltpu.sync_copy(sbuf, s_sh.at[pl.ds(r0 + j * K, K)])
        pltpu.sync_copy(eler[r][0], el_l)
        pltpu.sync_copy(eler[r][1], er_l)
        plsc.subcore_barrier()

        src_h, dst_h = edges[r]
        fs_h = fs_in[r]

        def _chunk(ci, _):
            base = wid * EW + ci * K
            pltpu.sync_copy(src_h.at[pl.ds(base, K)], srcb)
            pltpu.sync_copy(dst_h.at[pl.ds(base, K)], dstb)
            for i in range(K // 16):
                sv = srcb[pl.ds(i * 16, 16)]
                dv = dstb[pl.ds(i * 16, 16)]
                e = (plsc.load_gather(el_l, [sv]) +
                     plsc.load_gather(er_l, [dv]))
                e = jnp.where(e > 0, e, 0.2 * e)
                ex = jnp.exp(e)
                exb[pl.ds(i * 16, 16)] = ex
                plsc.store_scatter(sbuf, [iota + i * 16, zidx], ex)
            cp = pltpu.async_copy(fs_h.at[srcb], rowb, sem)
            cp.wait()

            def _scale(k, _c):
                exk = plsc.load_gather(exb, [zidx + k])
                for j in range(8):
                    rowb[k, pl.ds(j * 16, 16)] = (
                        rowb[k, pl.ds(j * 16, 16)] * exk)
                return _c
            lax.fori_loop(0, K, _scale, 0)

            pltpu.sync_copy(rowb, u_sh.at[dstb], add=True)
            pltpu.sync_copy(sbuf, s_sh.at[dstb], add=True)
            return _
        lax.fori_loop(0, NCH, _chunk, 0)

        plsc.subcore_barrier()
        u_h, s_h = outs[r]
        for j in range(RPT // 128):
            pltpu.sync_copy(u_sh.at[pl.ds(r0 + j * 128, 128)],
                            u_h.at[c, pl.ds(r0 + j * 128, 128)])
        pltpu.sync_copy(s_sh.at[pl.ds(r0, RPT)],
                        s_h.at[c, pl.ds(r0, RPT)])


def _sc_edge(fs0, fs1, el0, er0, el1, er1, src0, dst0, src1, dst1):
    mesh = plsc.VectorSubcoreMesh(core_axis_name="c", subcore_axis_name="s")
    fn = pl.kernel(
        _sc_edge_body,
        mesh=mesh,
        compiler_params=pltpu.CompilerParams(
            needs_layout_passes=False, use_tc_tiling_on_sc=False),
        out_type=[
            jax.ShapeDtypeStruct((2, NP, D), _f32),
            jax.ShapeDtypeStruct((2, NP, 16), _f32),
            jax.ShapeDtypeStruct((2, NP, D), _f32),
            jax.ShapeDtypeStruct((2, NP, 16), _f32),
        ],
        scratch_types=[
            pltpu.VMEM((NP,), _f32),
            pltpu.VMEM((NP,), _f32),
            pltpu.VMEM((K,), _i32),
            pltpu.VMEM((K,), _i32),
            pltpu.VMEM((K,), _f32),
            pltpu.VMEM((K, 16), _f32),
            pltpu.VMEM((K, D), _f32),
            pltpu.VMEM_SHARED((NP, D), _f32),
            pltpu.VMEM_SHARED((NP, 16), _f32),
            pltpu.SemaphoreType.DMA,
        ],
    )
    return fn(fs0, fs1, el0, er0, el1, er1, src0, dst0, src1, dst1)


def _norm_body(do_relu, u0_ref, s0_ref, u1_ref, s1_ref, bv_ref, g_ref, b_ref,
               x_ref):
    acc = None
    for (u_ref, s_ref) in ((u0_ref, s0_ref), (u1_ref, s1_ref)):
        u = u_ref[0] + u_ref[1]
        sv = jnp.sum(s_ref[0], axis=1) + jnp.sum(s_ref[1], axis=1)
        safe = jnp.where(sv > 0, sv, 1.0)
        t = u / safe[:, None]
        acc = t if acc is None else acc + t
    acc = acc + bv_ref[0][None, :]
    if do_relu:
        acc = jnp.maximum(acc, 0.0)
    mu = jnp.mean(acc, axis=1, keepdims=True)
    var = jnp.mean((acc - mu) ** 2, axis=1, keepdims=True)
    x_ref[...] = ((acc - mu) * lax.rsqrt(var + 1e-5) * g_ref[0][None, :]
                  + b_ref[0][None, :])


def _tc_norm(u0, s0, u1, s1, bv, g, b, do_relu):
    B = 1024
    grid = NP // B
    return pl.pallas_call(
        functools.partial(_norm_body, do_relu),
        grid=(grid,),
        in_specs=[
            pl.BlockSpec((2, B, D), lambda i: (0, i, 0)),
            pl.BlockSpec((2, B, 16), lambda i: (0, i, 0)),
            pl.BlockSpec((2, B, D), lambda i: (0, i, 0)),
            pl.BlockSpec((2, B, 16), lambda i: (0, i, 0)),
            pl.BlockSpec((1, D), lambda i: (0, 0)),
            pl.BlockSpec((1, D), lambda i: (0, 0)),
            pl.BlockSpec((1, D), lambda i: (0, 0)),
        ],
        out_specs=pl.BlockSpec((B, D), lambda i: (i, 0)),
        out_shape=jax.ShapeDtypeStruct((NP, D), _f32),
    )(u0, s0, u1, s1, bv, g, b)


def kernel(h, edge_index, Wsrc, Wdst, attn_l, attn_r, gat_bias, bias, ln_g, ln_b):
    x = jnp.pad(h, ((0, NP - N), (0, 0)))
    src0 = edge_index[0, 0]
    dst0 = edge_index[0, 1]
    src1 = edge_index[1, 0]
    dst1 = edge_index[1, 1]
    for l in range(3):
        fs0, fs1, el0, el1, er0, er1 = _tc_matmul(
            x, Wsrc[l], Wdst[l], attn_l[l], attn_r[l])
        u0, s0, u1, s1 = _sc_edge(
            fs0, fs1, el0, er0, el1, er1, src0, dst0, src1, dst1)
        bv = (bias[l] + gat_bias[l, 0] + gat_bias[l, 1])[None, :]
        x = _tc_norm(u0, s0, u1, s1, bv, ln_g[l][None, :], ln_b[l][None, :],
                     do_relu=(l < 2))
    return x[:N]
